# Initial kernel scaffold; baseline (speedup 1.0000x reference)
import jax
import jax.numpy as jnp
from jax.experimental import pallas as pl


def kernel(input_, weight):
    raise NotImplementedError("write your pallas kernel here")


# speedup vs baseline: 4.9458x; 4.9458x over previous
import functools

import jax
import jax.numpy as jnp
from jax import lax
from jax.experimental import pallas as pl
from jax.experimental.pallas import tpu as pltpu
from jax.experimental.pallas import tpu_sc as plsc

NUM_EMB = 1000000
DIM = 32
BATCH = 16384
HIST = 200
B_TOTAL = BATCH * HIST
NW = 32
B_PER_W = B_TOTAL // NW
CHUNK = 2048
N_CHUNKS = B_PER_W // CHUNK


def _emb_body(idx_hbm, table_hbm, out_hbm, idx_v, rows_v, sem):
    c = lax.axis_index("c")
    s = lax.axis_index("s")
    wid = s * 2 + c
    base = wid * B_PER_W

    def body(g, carry):
        off = base + g * CHUNK
        pltpu.sync_copy(idx_hbm.at[pl.ds(off, CHUNK)], idx_v)
        pltpu.async_copy(table_hbm.at[idx_v], rows_v, sem).wait()
        pltpu.sync_copy(rows_v, out_hbm.at[pl.ds(off, CHUNK)])
        return carry

    lax.fori_loop(0, N_CHUNKS, body, 0)


@jax.jit
def _embedding_sc(idx_flat, weight):
    mesh = plsc.VectorSubcoreMesh(core_axis_name="c", subcore_axis_name="s")
    f = pl.kernel(
        _emb_body,
        mesh=mesh,
        out_type=jax.ShapeDtypeStruct((B_TOTAL, DIM), jnp.float32),
        scratch_types=[
            pltpu.VMEM((CHUNK,), jnp.int32),
            pltpu.VMEM((CHUNK, DIM), jnp.float32),
            pltpu.SemaphoreType.DMA,
        ],
        compiler_params=pltpu.CompilerParams(use_tc_tiling_on_sc=False),
    )
    return f(idx_flat, weight)


def kernel(input_, weight):
    idx_flat = input_.reshape(-1).astype(jnp.int32)
    out = _embedding_sc(idx_flat, weight)
    return out.reshape(BATCH, HIST, DIM)


# speedup vs baseline: 5.0056x; 1.0121x over previous
import functools

import jax
import jax.numpy as jnp
from jax import lax
from jax.experimental import pallas as pl
from jax.experimental.pallas import tpu as pltpu
from jax.experimental.pallas import tpu_sc as plsc

NUM_EMB = 1000000
DIM = 32
BATCH = 16384
HIST = 200
B_TOTAL = BATCH * HIST
NW = 32
B_PER_W = B_TOTAL // NW
CHUNK = 1600
N_CHUNKS = B_PER_W // CHUNK


def _emb_body(idx_hbm, table_hbm, out_hbm, idx_v, rows_v, sem_g, sem_o):
    c = lax.axis_index("c")
    s = lax.axis_index("s")
    wid = s * 2 + c
    base = wid * B_PER_W

    def body(g, carry):
        b = lax.rem(g, 2)
        off = base + g * CHUNK
        pltpu.sync_copy(idx_hbm.at[pl.ds(off, CHUNK)], idx_v)
        @pl.when(g >= 2)
        def _():
            pltpu.make_async_copy(
                rows_v.at[b], out_hbm.at[pl.ds(off, CHUNK)], sem_o.at[b]
            ).wait()
        pltpu.async_copy(table_hbm.at[idx_v], rows_v.at[b], sem_g).wait()
        pltpu.make_async_copy(
            rows_v.at[b], out_hbm.at[pl.ds(off, CHUNK)], sem_o.at[b]
        ).start()
        return carry

    lax.fori_loop(0, N_CHUNKS, body, 0)
    for g in (N_CHUNKS - 2, N_CHUNKS - 1):
        b = g % 2
        off = base + g * CHUNK
        pltpu.make_async_copy(
            rows_v.at[b], out_hbm.at[pl.ds(off, CHUNK)], sem_o.at[b]
        ).wait()


@jax.jit
def _embedding_sc(idx_flat, weight):
    mesh = plsc.VectorSubcoreMesh(core_axis_name="c", subcore_axis_name="s")
    f = pl.kernel(
        _emb_body,
        mesh=mesh,
        out_type=jax.ShapeDtypeStruct((B_TOTAL, DIM), jnp.float32),
        scratch_types=[
            pltpu.VMEM((CHUNK,), jnp.int32),
            pltpu.VMEM((2, CHUNK, DIM), jnp.float32),
            pltpu.SemaphoreType.DMA,
            pltpu.SemaphoreType.DMA((2,)),
        ],
        compiler_params=pltpu.CompilerParams(use_tc_tiling_on_sc=False),
    )
    return f(idx_flat, weight)


def kernel(input_, weight):
    idx_flat = input_.reshape(-1).astype(jnp.int32)
    out = _embedding_sc(idx_flat, weight)
    return out.reshape(BATCH, HIST, DIM)


# speedup vs baseline: 5.0463x; 1.0081x over previous
import functools

import jax
import jax.numpy as jnp
from jax import lax
from jax.experimental import pallas as pl
from jax.experimental.pallas import tpu as pltpu
from jax.experimental.pallas import tpu_sc as plsc

NUM_EMB = 1000000
DIM = 32
BATCH = 16384
HIST = 200
B_TOTAL = BATCH * HIST
NW = 32
B_PER_W = B_TOTAL // NW
CHUNK = 1600
N_CHUNKS = B_PER_W // CHUNK


def _emb_body(idx_hbm, table_hbm, out_hbm, idx_v, rows_v, sem_i, sem_g, sem_o):
    c = lax.axis_index("c")
    s = lax.axis_index("s")
    wid = s * 2 + c
    base = wid * B_PER_W

    def idx_copy(g, b):
        return pltpu.make_async_copy(
            idx_hbm.at[pl.ds(base + g * CHUNK, CHUNK)], idx_v.at[b], sem_i.at[b]
        )

    def gather(b):
        return pltpu.make_async_copy(table_hbm.at[idx_v.at[b]], rows_v.at[b],
                                     sem_g.at[b])

    def writeback(g, b):
        return pltpu.make_async_copy(
            rows_v.at[b], out_hbm.at[pl.ds(base + g * CHUNK, CHUNK)], sem_o.at[b]
        )

    idx_copy(0, 0).start()
    idx_copy(1, 1).start()
    idx_copy(0, 0).wait()
    gather(0).start()

    def body(g, carry):
        b = lax.rem(g, 2)
        nb = lax.rem(g + 1, 2)

        @pl.when(g + 1 < N_CHUNKS)
        def _():
            idx_copy(g + 1, nb).wait()

            @pl.when(g >= 1)
            def _():
                writeback(g - 1, nb).wait()

            gather(nb).start()

        pltpu.make_async_copy(table_hbm.at[idx_v.at[b]], rows_v.at[b],
                              sem_g.at[b]).wait()

        @pl.when(g + 2 < N_CHUNKS)
        def _():
            idx_copy(g + 2, b).start()

        writeback(g, b).start()
        return carry

    lax.fori_loop(0, N_CHUNKS, body, 0)
    for g in (N_CHUNKS - 2, N_CHUNKS - 1):
        writeback(g, g % 2).wait()


@jax.jit
def _embedding_sc(idx_flat, weight):
    mesh = plsc.VectorSubcoreMesh(core_axis_name="c", subcore_axis_name="s")
    f = pl.kernel(
        _emb_body,
        mesh=mesh,
        out_type=jax.ShapeDtypeStruct((B_TOTAL, DIM), jnp.float32),
        scratch_types=[
            pltpu.VMEM((2, CHUNK), jnp.int32),
            pltpu.VMEM((2, CHUNK, DIM), jnp.float32),
            pltpu.SemaphoreType.DMA((2,)),
            pltpu.SemaphoreType.DMA((2,)),
            pltpu.SemaphoreType.DMA((2,)),
        ],
        compiler_params=pltpu.CompilerParams(use_tc_tiling_on_sc=False),
    )
    return f(idx_flat, weight)


def kernel(input_, weight):
    idx_flat = input_.reshape(-1).astype(jnp.int32)
    out = _embedding_sc(idx_flat, weight)
    return out.reshape(BATCH, HIST, DIM)
